# Initial kernel scaffold; baseline (speedup 1.0000x reference)
import jax
import jax.numpy as jnp
from jax.experimental import pallas as pl


def kernel(x, edge_index, edge_attr, batch, graph_attr, W_node, b_node, W_edge, b_edge, W11, b11, W12, b12, W21, b21, W22, b22, g1, be1, g2, be2, Wfc, bfc):
    raise NotImplementedError("write your pallas kernel here")


# speedup vs baseline: 2.3207x; 2.3207x over previous
import functools

import jax
import jax.numpy as jnp
from jax import lax
from jax.experimental import pallas as pl
from jax.experimental.pallas import tpu as pltpu
from jax.experimental.pallas import tpu_sc as plsc

N = 10000
E = 160000
H = 512
C = 4
HC = H // C
NB = 1000
EB = 2000
NT = 16
NCORES = 2
K = 80
EPT = E // NT
GCH = EPT // K
NPAD = 10240
NPT = NPAD // NT
ZR = 32


def _node_embed_body(x_ref, w_ref, b_ref, o0, o1, o2, o3):
    xb = x_ref[...]
    for c, o in enumerate((o0, o1, o2, o3)):
        acc = jnp.dot(xb, w_ref[:, c, :], preferred_element_type=jnp.float32)
        o[...] = jnp.maximum(acc + b_ref[c:c + 1, :], 0.0)


def _node_embed(x, w4, b4):
    fdim = x.shape[1]
    return pl.pallas_call(
        _node_embed_body,
        grid=(N // NB,),
        in_specs=[
            pl.BlockSpec((NB, fdim), lambda i: (i, 0)),
            pl.BlockSpec((fdim, C, HC), lambda i: (0, 0, 0)),
            pl.BlockSpec((C, HC), lambda i: (0, 0)),
        ],
        out_specs=[pl.BlockSpec((NB, HC), lambda i: (i, 0))] * C,
        out_shape=[jax.ShapeDtypeStruct((N, HC), jnp.float32)] * C,
    )(x, w4, b4)


def _edge_embed_body(a_ref, w_ref, b_ref, o0, o1, o2, o3):
    ab = a_ref[...]
    for c, o in enumerate((o0, o1, o2, o3)):
        acc = jnp.dot(ab, w_ref[:, c, :], preferred_element_type=jnp.float32)
        o[...] = jnp.maximum(acc + b_ref[c:c + 1, :], 0.0)


def _edge_embed(ea, w4, b4):
    fdim = ea.shape[1]
    return pl.pallas_call(
        _edge_embed_body,
        grid=(E // EB,),
        in_specs=[
            pl.BlockSpec((EB, fdim), lambda i: (i, 0)),
            pl.BlockSpec((fdim, C, HC), lambda i: (0, 0, 0)),
            pl.BlockSpec((C, HC), lambda i: (0, 0)),
        ],
        out_specs=[pl.BlockSpec((EB, HC), lambda i: (i, 0))] * C,
        out_shape=[jax.ShapeDtypeStruct((E, HC), jnp.float32)] * C,
    )(ea, w4, b4)


def _sc_agg_body(h0, h1, h2, h3, e0, e1, e2, e3, sdr,
                 a0, a1, a2, a3,
                 idx_b, hbuf, ebuf, zbuf, acc, sem_h, sem_e):
    cid = lax.axis_index("c")
    sid = lax.axis_index("s")
    htabs = (h0, h1, h2, h3)
    etabs = (e0, e1, e2, e3)
    atabs = (a0, a1, a2, a3)

    z16 = jnp.zeros((16,), jnp.float32)

    def _zfill(i, _):
        r = i // (HC // 16)
        j = (i % (HC // 16)) * 16
        zbuf[r, pl.ds(j, 16)] = z16
        return 0

    lax.fori_loop(0, ZR * (HC // 16), _zfill, 0)

    ebase = sid * EPT

    for p in range(2):
        for z in range(NPT // ZR):
            pltpu.sync_copy(zbuf, acc.at[pl.ds(sid * NPT + z * ZR, ZR)])
        plsc.subcore_barrier()

        for cc in range(NCORES):
            ch = NCORES * p + cc

            @pl.when(cid == cc)
            def _(ch=ch):
                htab = htabs[ch]
                etab = etabs[ch]

                def _edge_chunk(g, _):
                    pltpu.sync_copy(sdr.at[sid, g], idx_b)
                    cph = pltpu.async_copy(
                        htab.at[idx_b.at[0]], hbuf, sem_h)
                    cpe = pltpu.async_copy(
                        etab.at[pl.ds(ebase + g * K, K)], ebuf, sem_e)
                    cph.wait()
                    cpe.wait()

                    def _row(r, _):
                        for j in range(HC // 16):
                            sl = pl.ds(j * 16, 16)
                            v = hbuf[r, sl] + ebuf[r, sl]
                            hbuf[r, sl] = jnp.maximum(v, 0.0)
                        return 0

                    lax.fori_loop(0, K, _row, 0)
                    pltpu.sync_copy(hbuf, acc.at[idx_b.at[1]], add=True)
                    return 0

                lax.fori_loop(0, GCH, _edge_chunk, 0)

        plsc.subcore_barrier()

        for cc in range(NCORES):
            ch = NCORES * p + cc

            @pl.when(cid == cc)
            def _(ch=ch):
                sl = pl.ds(sid * NPT, NPT)
                pltpu.sync_copy(acc.at[sl], atabs[ch].at[sl])

        if p == 0:
            plsc.subcore_barrier()


def _sc_agg(hc, ec, sdr):
    f = pl.kernel(
        _sc_agg_body,
        out_type=[jax.ShapeDtypeStruct((NPAD, HC), jnp.float32)] * C,
        mesh=plsc.VectorSubcoreMesh(
            core_axis_name="c", subcore_axis_name="s",
            num_cores=NCORES, num_subcores=NT),
        scratch_types=[
            pltpu.VMEM((2, K), jnp.int32),
            pltpu.VMEM((K, HC), jnp.float32),
            pltpu.VMEM((K, HC), jnp.float32),
            pltpu.VMEM((ZR, HC), jnp.float32),
            pltpu.VMEM_SHARED((NPAD, HC), jnp.float32),
            pltpu.SemaphoreType.DMA,
            pltpu.SemaphoreType.DMA,
        ],
    )
    return f(*hc, *ec, sdr)


def _mlp_body(h0, h1, h2, h3, a0, a1, a2, a3,
              wa_ref, ba_ref, wb_ref, bb_ref,
              y0, y1, y2, y3, st_ref, acc_ref):
    i = pl.program_id(0)

    @pl.when(i == 0)
    def _():
        acc_ref[...] = jnp.zeros_like(acc_ref)

    hs = (h0, h1, h2, h3)
    As = (a0, a1, a2, a3)
    z = None
    for c in range(C):
        inb = hs[c][...] + As[c][...]
        part = jnp.dot(inb, wa_ref[c * HC:(c + 1) * HC, :],
                       preferred_element_type=jnp.float32)
        z = part if z is None else z + part
    t = jnp.maximum(z + ba_ref[...], 0.0)

    ys = (y0, y1, y2, y3)
    for c in range(C):
        cs = slice(c * HC, (c + 1) * HC)
        yc = jnp.dot(t, wb_ref[:, cs], preferred_element_type=jnp.float32)
        yc = yc + bb_ref[:, cs]
        ys[c][...] = yc
        acc_ref[0:1, cs] += jnp.sum(yc, axis=0, keepdims=True)
        acc_ref[1:2, cs] += jnp.sum(yc * yc, axis=0, keepdims=True)

    st_ref[...] = acc_ref[...]


def _mlp(hc, ac, wa, ba, wb, bb):
    outs = pl.pallas_call(
        _mlp_body,
        grid=(N // NB,),
        in_specs=(
            [pl.BlockSpec((NB, HC), lambda i: (i, 0))] * (2 * C)
            + [
                pl.BlockSpec((H, H), lambda i: (0, 0)),
                pl.BlockSpec((1, H), lambda i: (0, 0)),
                pl.BlockSpec((H, H), lambda i: (0, 0)),
                pl.BlockSpec((1, H), lambda i: (0, 0)),
            ]
        ),
        out_specs=(
            [pl.BlockSpec((NB, HC), lambda i: (i, 0))] * C
            + [pl.BlockSpec((8, H), lambda i: (0, 0))]
        ),
        out_shape=(
            [jax.ShapeDtypeStruct((N, HC), jnp.float32)] * C
            + [jax.ShapeDtypeStruct((8, H), jnp.float32)]
        ),
        scratch_shapes=[pltpu.VMEM((8, H), jnp.float32)],
    )(*hc, *ac, wa, ba, wb, bb)
    return outs[:C], outs[C]


def _bnrelu_body(y0, y1, y2, y3, st_ref, g_ref, b_ref, o0, o1, o2, o3):
    m = st_ref[0:1, :] * (1.0 / N)
    q = st_ref[1:2, :] * (1.0 / N)
    inv = lax.rsqrt(q - m * m + 1e-5)
    ys = (y0, y1, y2, y3)
    os_ = (o0, o1, o2, o3)
    for c in range(C):
        cs = slice(c * HC, (c + 1) * HC)
        hn = (ys[c][...] - m[:, cs]) * (inv[:, cs] * g_ref[:, cs]) + b_ref[:, cs]
        os_[c][...] = jnp.maximum(hn, 0.0)


def _bnrelu(yc, st, g, b):
    return pl.pallas_call(
        _bnrelu_body,
        grid=(N // NB,),
        in_specs=(
            [pl.BlockSpec((NB, HC), lambda i: (i, 0))] * C
            + [
                pl.BlockSpec((8, H), lambda i: (0, 0)),
                pl.BlockSpec((1, H), lambda i: (0, 0)),
                pl.BlockSpec((1, H), lambda i: (0, 0)),
            ]
        ),
        out_specs=[pl.BlockSpec((NB, HC), lambda i: (i, 0))] * C,
        out_shape=[jax.ShapeDtypeStruct((N, HC), jnp.float32)] * C,
    )(*yc, st, g, b)


def _pool_body(y0, y1, y2, y3, st_ref, g_ref, b_ref, batch_ref, ga_ref,
               wh_ref, wg_ref, bfc_ref, out_ref, accp_ref, accc_ref):
    i = pl.program_id(0)

    @pl.when(i == 0)
    def _():
        accp_ref[...] = jnp.zeros_like(accp_ref)
        accc_ref[...] = jnp.zeros_like(accc_ref)

    m = st_ref[0:1, :] * (1.0 / N)
    q = st_ref[1:2, :] * (1.0 / N)
    inv = lax.rsqrt(q - m * m + 1e-5)

    bi = batch_ref[0]
    oh = (bi == lax.broadcasted_iota(jnp.int32, (64, NB), 0))
    ohf = oh.astype(jnp.float32)
    cnt = jnp.sum(ohf, axis=1, keepdims=True)
    accc_ref[...] += jnp.broadcast_to(cnt, (64, 128))

    ys = (y0, y1, y2, y3)
    for c in range(C):
        cs = slice(c * HC, (c + 1) * HC)
        hn = (ys[c][...] - m[:, cs]) * (inv[:, cs] * g_ref[:, cs]) + b_ref[:, cs]
        hn = jnp.maximum(hn, 0.0)
        accp_ref[:, cs] += lax.dot_general(
            ohf, hn, (((1,), (0,)), ((), ())),
            preferred_element_type=jnp.float32)

    @pl.when(i == pl.num_programs(0) - 1)
    def _():
        cnt_all = jnp.maximum(accc_ref[:, 0:1], 1.0)
        pooled = accp_ref[...] / cnt_all
        r = jnp.dot(pooled, wh_ref[...], preferred_element_type=jnp.float32)
        r = r + jnp.dot(ga_ref[...], wg_ref[...],
                        preferred_element_type=jnp.float32)
        out_ref[...] = r + bfc_ref[...]


def _pool(yc, st, g, b, batch3, ga, wh, wg, bfc):
    return pl.pallas_call(
        _pool_body,
        grid=(N // NB,),
        in_specs=(
            [pl.BlockSpec((NB, HC), lambda i: (i, 0))] * C
            + [
                pl.BlockSpec((8, H), lambda i: (0, 0)),
                pl.BlockSpec((1, H), lambda i: (0, 0)),
                pl.BlockSpec((1, H), lambda i: (0, 0)),
                pl.BlockSpec((1, 1, NB), lambda i: (i, 0, 0)),
                pl.BlockSpec((64, 10), lambda i: (0, 0)),
                pl.BlockSpec((H, 3), lambda i: (0, 0)),
                pl.BlockSpec((10, 3), lambda i: (0, 0)),
                pl.BlockSpec((1, 3), lambda i: (0, 0)),
            ]
        ),
        out_specs=pl.BlockSpec((64, 3), lambda i: (0, 0)),
        out_shape=jax.ShapeDtypeStruct((64, 3), jnp.float32),
        scratch_shapes=[
            pltpu.VMEM((64, H), jnp.float32),
            pltpu.VMEM((64, 128), jnp.float32),
        ],
    )(*yc, st, g, b, batch3, ga, wh, wg, bfc)


def kernel(x, edge_index, edge_attr, batch, graph_attr,
           W_node, b_node, W_edge, b_edge,
           W11, b11, W12, b12, W21, b21, W22, b22,
           g1, be1, g2, be2, Wfc, bfc):
    sdr = jnp.stack([edge_index[0].reshape(NT, GCH, K),
                     edge_index[1].reshape(NT, GCH, K)], axis=2)

    hc0 = _node_embed(x, W_node.reshape(x.shape[1], C, HC),
                      b_node.reshape(C, HC))
    ec = _edge_embed(edge_attr, W_edge.reshape(edge_attr.shape[1], C, HC),
                     b_edge.reshape(C, HC))

    a1 = _sc_agg(hc0, ec, sdr)
    y1, st1 = _mlp(hc0, a1, W11, b11.reshape(1, H), W12, b12.reshape(1, H))
    hc1 = _bnrelu(y1, st1, g1.reshape(1, H), be1.reshape(1, H))

    a2 = _sc_agg(hc1, ec, sdr)
    y2, st2 = _mlp(hc1, a2, W21, b21.reshape(1, H), W22, b22.reshape(1, H))

    return _pool(y2, st2, g2.reshape(1, H), be2.reshape(1, H),
                 batch.reshape(N // NB, 1, NB), graph_attr,
                 Wfc[:H], Wfc[H:], bfc.reshape(1, 3))


# speedup vs baseline: 3.5327x; 1.5223x over previous
import functools

import jax
import jax.numpy as jnp
from jax import lax
from jax.experimental import pallas as pl
from jax.experimental.pallas import tpu as pltpu
from jax.experimental.pallas import tpu_sc as plsc

N = 10000
E = 160000
H = 512
C = 4
HC = H // C
NB = 1000
EB = 2000
NT = 16
NCORES = 2
K = 40
EPT = E // NT
GCH = EPT // K
NPAD = 10240
NPT = NPAD // NT
ZR = 32


def _node_embed_body(x_ref, w_ref, b_ref, o0, o1, o2, o3):
    xb = x_ref[...]
    for c, o in enumerate((o0, o1, o2, o3)):
        acc = jnp.dot(xb, w_ref[:, c, :], preferred_element_type=jnp.float32)
        o[...] = jnp.maximum(acc + b_ref[c:c + 1, :], 0.0)


def _node_embed(x, w4, b4):
    fdim = x.shape[1]
    return pl.pallas_call(
        _node_embed_body,
        grid=(N // NB,),
        in_specs=[
            pl.BlockSpec((NB, fdim), lambda i: (i, 0)),
            pl.BlockSpec((fdim, C, HC), lambda i: (0, 0, 0)),
            pl.BlockSpec((C, HC), lambda i: (0, 0)),
        ],
        out_specs=[pl.BlockSpec((NB, HC), lambda i: (i, 0))] * C,
        out_shape=[jax.ShapeDtypeStruct((N, HC), jnp.float32)] * C,
    )(x, w4, b4)


def _edge_embed_body(a_ref, w_ref, b_ref, o0, o1, o2, o3):
    ab = a_ref[...]
    for c, o in enumerate((o0, o1, o2, o3)):
        acc = jnp.dot(ab, w_ref[:, c, :], preferred_element_type=jnp.float32)
        o[...] = jnp.maximum(acc + b_ref[c:c + 1, :], 0.0)


def _edge_embed(ea, w4, b4):
    fdim = ea.shape[1]
    return pl.pallas_call(
        _edge_embed_body,
        grid=(E // EB,),
        in_specs=[
            pl.BlockSpec((EB, fdim), lambda i: (i, 0)),
            pl.BlockSpec((fdim, C, HC), lambda i: (0, 0, 0)),
            pl.BlockSpec((C, HC), lambda i: (0, 0)),
        ],
        out_specs=[pl.BlockSpec((EB, HC), lambda i: (i, 0))] * C,
        out_shape=[jax.ShapeDtypeStruct((E, HC), jnp.float32)] * C,
    )(ea, w4, b4)


PAIRS = 125


def _sc_agg_body(h0, h1, h2, h3, e0, e1, e2, e3, sdr,
                 a0, a1, a2, a3,
                 ig0, ig1, is0, is1, hb0, hb1, eb0, eb1, mb0, mb1,
                 zbuf, acc,
                 sh0, sh1, se0, se1, si0, si1, sj0, sj1, sw0, sw1):
    cid = lax.axis_index("c")
    sid = lax.axis_index("s")
    htabs = (h0, h1, h2, h3)
    etabs = (e0, e1, e2, e3)
    atabs = (a0, a1, a2, a3)
    idxg = (ig0, ig1)
    idxs = (is0, is1)
    hb = (hb0, hb1)
    eb = (eb0, eb1)
    mb = (mb0, mb1)
    sh = (sh0, sh1)
    se = (se0, se1)
    si = (si0, si1)
    sj = (sj0, sj1)
    sw = (sw0, sw1)

    z16 = jnp.zeros((16,), jnp.float32)

    def _zfill(i, _):
        r = i // (HC // 16)
        j = (i % (HC // 16)) * 16
        zbuf[r, pl.ds(j, 16)] = z16
        return 0

    lax.fori_loop(0, ZR * (HC // 16), _zfill, 0)

    ebase = sid * EPT

    for p in range(2):
        for z in range(NPT // ZR):
            pltpu.sync_copy(zbuf, acc.at[pl.ds(sid * NPT + z * ZR, ZR)])
        plsc.subcore_barrier()

        for cc in range(NCORES):
            ch = NCORES * p + cc

            @pl.when(cid == cc)
            def _(ch=ch):
                htab = htabs[ch]
                etab = etabs[ch]

                for s in range(2):
                    pltpu.sync_copy(sdr.at[sid, s], idxg[s])
                    pltpu.async_copy(htab.at[idxg[s].at[0]], hb[s], sh[s])
                    pltpu.async_copy(
                        etab.at[pl.ds(ebase + s * K, K)], eb[s], se[s])

                def _pair(t, _):
                    for s in range(2):
                        g = 2 * t + s
                        pltpu.make_async_copy(
                            etab.at[pl.ds(0, K)], hb[s], sh[s]).wait()
                        pltpu.make_async_copy(
                            etab.at[pl.ds(0, K)], eb[s], se[s]).wait()

                        @pl.when(t > 0)
                        def _():
                            pltpu.make_async_copy(
                                etab.at[pl.ds(0, K)], mb[s], sw[s]).wait()

                        @pl.when(t < PAIRS - 1)
                        def _():
                            pltpu.async_copy(sdr.at[sid, g + 2],
                                             idxg[s], si[s])
                        pltpu.async_copy(sdr.at[sid, g], idxs[s], sj[s])

                        def _row(r, _):
                            for j in range(HC // 16):
                                sl = pl.ds(j * 16, 16)
                                v = hb[s][r, sl] + eb[s][r, sl]
                                mb[s][r, sl] = jnp.maximum(v, 0.0)
                            return 0

                        lax.fori_loop(0, K, _row, 0)

                        pltpu.make_async_copy(
                            sdr.at[sid, g], idxs[s], sj[s]).wait()
                        pltpu.async_copy(mb[s], acc.at[idxs[s].at[1]],
                                         sw[s], add=True)

                        @pl.when(t < PAIRS - 1)
                        def _():
                            pltpu.make_async_copy(
                                sdr.at[sid, g], idxg[s], si[s]).wait()
                            pltpu.async_copy(
                                htab.at[idxg[s].at[0]], hb[s], sh[s])
                            pltpu.async_copy(
                                etab.at[pl.ds(ebase + (g + 2) * K, K)],
                                eb[s], se[s])
                    return 0

                lax.fori_loop(0, PAIRS, _pair, 0)

                for s in range(2):
                    pltpu.make_async_copy(
                        etab.at[pl.ds(0, K)], mb[s], sw[s]).wait()

        plsc.subcore_barrier()

        for cc in range(NCORES):
            ch = NCORES * p + cc

            @pl.when(cid == cc)
            def _(ch=ch):
                sl = pl.ds(sid * NPT, NPT)
                pltpu.sync_copy(acc.at[sl], atabs[ch].at[sl])

        if p == 0:
            plsc.subcore_barrier()


def _sc_agg(hc, ec, sdr):
    f = pl.kernel(
        _sc_agg_body,
        out_type=[jax.ShapeDtypeStruct((NPAD, HC), jnp.float32)] * C,
        mesh=plsc.VectorSubcoreMesh(
            core_axis_name="c", subcore_axis_name="s",
            num_cores=NCORES, num_subcores=NT),
        scratch_types=(
            [pltpu.VMEM((2, K), jnp.int32)] * 4
            + [pltpu.VMEM((K, HC), jnp.float32)] * 6
            + [
                pltpu.VMEM((ZR, HC), jnp.float32),
                pltpu.VMEM_SHARED((NPAD, HC), jnp.float32),
            ]
            + [pltpu.SemaphoreType.DMA] * 10
        ),
    )
    return f(*hc, *ec, sdr)


def _mlp_body(h0, h1, h2, h3, a0, a1, a2, a3,
              wa_ref, ba_ref, wb_ref, bb_ref,
              y0, y1, y2, y3, st_ref, acc_ref):
    i = pl.program_id(0)

    @pl.when(i == 0)
    def _():
        acc_ref[...] = jnp.zeros_like(acc_ref)

    hs = (h0, h1, h2, h3)
    As = (a0, a1, a2, a3)
    z = None
    for c in range(C):
        inb = hs[c][...] + As[c][...]
        part = jnp.dot(inb, wa_ref[c * HC:(c + 1) * HC, :],
                       preferred_element_type=jnp.float32)
        z = part if z is None else z + part
    t = jnp.maximum(z + ba_ref[...], 0.0)

    ys = (y0, y1, y2, y3)
    for c in range(C):
        cs = slice(c * HC, (c + 1) * HC)
        yc = jnp.dot(t, wb_ref[:, cs], preferred_element_type=jnp.float32)
        yc = yc + bb_ref[:, cs]
        ys[c][...] = yc
        acc_ref[0:1, cs] += jnp.sum(yc, axis=0, keepdims=True)
        acc_ref[1:2, cs] += jnp.sum(yc * yc, axis=0, keepdims=True)

    st_ref[...] = acc_ref[...]


def _mlp(hc, ac, wa, ba, wb, bb):
    outs = pl.pallas_call(
        _mlp_body,
        grid=(N // NB,),
        in_specs=(
            [pl.BlockSpec((NB, HC), lambda i: (i, 0))] * (2 * C)
            + [
                pl.BlockSpec((H, H), lambda i: (0, 0)),
                pl.BlockSpec((1, H), lambda i: (0, 0)),
                pl.BlockSpec((H, H), lambda i: (0, 0)),
                pl.BlockSpec((1, H), lambda i: (0, 0)),
            ]
        ),
        out_specs=(
            [pl.BlockSpec((NB, HC), lambda i: (i, 0))] * C
            + [pl.BlockSpec((8, H), lambda i: (0, 0))]
        ),
        out_shape=(
            [jax.ShapeDtypeStruct((N, HC), jnp.float32)] * C
            + [jax.ShapeDtypeStruct((8, H), jnp.float32)]
        ),
        scratch_shapes=[pltpu.VMEM((8, H), jnp.float32)],
    )(*hc, *ac, wa, ba, wb, bb)
    return outs[:C], outs[C]


def _bnrelu_body(y0, y1, y2, y3, st_ref, g_ref, b_ref, o0, o1, o2, o3):
    m = st_ref[0:1, :] * (1.0 / N)
    q = st_ref[1:2, :] * (1.0 / N)
    inv = lax.rsqrt(q - m * m + 1e-5)
    ys = (y0, y1, y2, y3)
    os_ = (o0, o1, o2, o3)
    for c in range(C):
        cs = slice(c * HC, (c + 1) * HC)
        hn = (ys[c][...] - m[:, cs]) * (inv[:, cs] * g_ref[:, cs]) + b_ref[:, cs]
        os_[c][...] = jnp.maximum(hn, 0.0)


def _bnrelu(yc, st, g, b):
    return pl.pallas_call(
        _bnrelu_body,
        grid=(N // NB,),
        in_specs=(
            [pl.BlockSpec((NB, HC), lambda i: (i, 0))] * C
            + [
                pl.BlockSpec((8, H), lambda i: (0, 0)),
                pl.BlockSpec((1, H), lambda i: (0, 0)),
                pl.BlockSpec((1, H), lambda i: (0, 0)),
            ]
        ),
        out_specs=[pl.BlockSpec((NB, HC), lambda i: (i, 0))] * C,
        out_shape=[jax.ShapeDtypeStruct((N, HC), jnp.float32)] * C,
    )(*yc, st, g, b)


def _pool_body(y0, y1, y2, y3, st_ref, g_ref, b_ref, batch_ref, ga_ref,
               wh_ref, wg_ref, bfc_ref, out_ref, accp_ref, accc_ref):
    i = pl.program_id(0)

    @pl.when(i == 0)
    def _():
        accp_ref[...] = jnp.zeros_like(accp_ref)
        accc_ref[...] = jnp.zeros_like(accc_ref)

    m = st_ref[0:1, :] * (1.0 / N)
    q = st_ref[1:2, :] * (1.0 / N)
    inv = lax.rsqrt(q - m * m + 1e-5)

    bi = batch_ref[0]
    oh = (bi == lax.broadcasted_iota(jnp.int32, (64, NB), 0))
    ohf = oh.astype(jnp.float32)
    cnt = jnp.sum(ohf, axis=1, keepdims=True)
    accc_ref[...] += jnp.broadcast_to(cnt, (64, 128))

    ys = (y0, y1, y2, y3)
    for c in range(C):
        cs = slice(c * HC, (c + 1) * HC)
        hn = (ys[c][...] - m[:, cs]) * (inv[:, cs] * g_ref[:, cs]) + b_ref[:, cs]
        hn = jnp.maximum(hn, 0.0)
        accp_ref[:, cs] += lax.dot_general(
            ohf, hn, (((1,), (0,)), ((), ())),
            preferred_element_type=jnp.float32)

    @pl.when(i == pl.num_programs(0) - 1)
    def _():
        cnt_all = jnp.maximum(accc_ref[:, 0:1], 1.0)
        pooled = accp_ref[...] / cnt_all
        r = jnp.dot(pooled, wh_ref[...], preferred_element_type=jnp.float32)
        r = r + jnp.dot(ga_ref[...], wg_ref[...],
                        preferred_element_type=jnp.float32)
        out_ref[...] = r + bfc_ref[...]


def _pool(yc, st, g, b, batch3, ga, wh, wg, bfc):
    return pl.pallas_call(
        _pool_body,
        grid=(N // NB,),
        in_specs=(
            [pl.BlockSpec((NB, HC), lambda i: (i, 0))] * C
            + [
                pl.BlockSpec((8, H), lambda i: (0, 0)),
                pl.BlockSpec((1, H), lambda i: (0, 0)),
                pl.BlockSpec((1, H), lambda i: (0, 0)),
                pl.BlockSpec((1, 1, NB), lambda i: (i, 0, 0)),
                pl.BlockSpec((64, 10), lambda i: (0, 0)),
                pl.BlockSpec((H, 3), lambda i: (0, 0)),
                pl.BlockSpec((10, 3), lambda i: (0, 0)),
                pl.BlockSpec((1, 3), lambda i: (0, 0)),
            ]
        ),
        out_specs=pl.BlockSpec((64, 3), lambda i: (0, 0)),
        out_shape=jax.ShapeDtypeStruct((64, 3), jnp.float32),
        scratch_shapes=[
            pltpu.VMEM((64, H), jnp.float32),
            pltpu.VMEM((64, 128), jnp.float32),
        ],
    )(*yc, st, g, b, batch3, ga, wh, wg, bfc)


def kernel(x, edge_index, edge_attr, batch, graph_attr,
           W_node, b_node, W_edge, b_edge,
           W11, b11, W12, b12, W21, b21, W22, b22,
           g1, be1, g2, be2, Wfc, bfc):
    sdr = jnp.stack([edge_index[0].reshape(NT, GCH, K),
                     edge_index[1].reshape(NT, GCH, K)], axis=2)

    hc0 = _node_embed(x, W_node.reshape(x.shape[1], C, HC),
                      b_node.reshape(C, HC))
    ec = _edge_embed(edge_attr, W_edge.reshape(edge_attr.shape[1], C, HC),
                     b_edge.reshape(C, HC))

    a1 = _sc_agg(hc0, ec, sdr)
    y1, st1 = _mlp(hc0, a1, W11, b11.reshape(1, H), W12, b12.reshape(1, H))
    hc1 = _bnrelu(y1, st1, g1.reshape(1, H), be1.reshape(1, H))

    a2 = _sc_agg(hc1, ec, sdr)
    y2, st2 = _mlp(hc1, a2, W21, b21.reshape(1, H), W22, b22.reshape(1, H))

    return _pool(y2, st2, g2.reshape(1, H), be2.reshape(1, H),
                 batch.reshape(N // NB, 1, NB), graph_attr,
                 Wfc[:H], Wfc[H:], bfc.reshape(1, 3))
